# Initial kernel scaffold; baseline (speedup 1.0000x reference)
import jax
import jax.numpy as jnp
from jax.experimental import pallas as pl


def kernel(x):
    raise NotImplementedError("write your pallas kernel here")


# speedup vs baseline: 9.5557x; 9.5557x over previous
import functools

import jax
import jax.numpy as jnp
import numpy as np
from jax.experimental import pallas as pl

_ROWS = 128
_COLS = 32768
_K = int(0.05 * _COLS)
_BLOCK_ROWS = 8
_SIGN = np.int32(np.uint32(0x80000000).view(np.int32))


def _kwta_body(x_ref, o_ref):
    x = x_ref[...]
    b = jax.lax.bitcast_convert_type(x, jnp.int32)
    keys = b ^ jax.lax.shift_right_logical(
        jax.lax.shift_right_arithmetic(b, 31), 1
    )
    r = x.shape[0]
    prefix_u = jnp.zeros((r, 1), jnp.int32)
    for bit in range(31, -1, -1):
        bitpat = np.int32(np.uint32(1 << bit).view(np.int32))
        trial_u = prefix_u | bitpat
        trial_s = trial_u ^ _SIGN
        cnt = jnp.sum((keys >= trial_s).astype(jnp.int32), axis=1,
                      keepdims=True)
        prefix_u = jnp.where(cnt >= _K, trial_u, prefix_u)
    thr_s = prefix_u ^ _SIGN
    o_ref[...] = jnp.where(keys >= thr_s, x, 0.0)


@jax.jit
def kernel(x):
    grid = (_ROWS // _BLOCK_ROWS,)
    return pl.pallas_call(
        _kwta_body,
        grid=grid,
        in_specs=[pl.BlockSpec((_BLOCK_ROWS, _COLS), lambda i: (i, 0))],
        out_specs=pl.BlockSpec((_BLOCK_ROWS, _COLS), lambda i: (i, 0)),
        out_shape=jax.ShapeDtypeStruct((_ROWS, _COLS), x.dtype),
    )(x)
